# Initial kernel scaffold; baseline (speedup 1.0000x reference)
import jax
import jax.numpy as jnp
from jax.experimental import pallas as pl


def kernel(x, pos, edge_index, W_feat, b_feat, g_feat, be_feat, W_wf, b_wf, g_wf, be_wf, W_q, b_q, W_k, b_k, W_pos, b_pos, g_pos, be_pos, g_fin, be_fin):
    raise NotImplementedError("write your pallas kernel here")


# speedup vs baseline: 3.4297x; 3.4297x over previous
import functools
import math

import jax
import jax.numpy as jnp
from jax import lax
from jax.experimental import pallas as pl
from jax.experimental.pallas import tpu as pltpu
import jax.experimental.pallas.tpu_sc as plsc

_NC = 2
_NS = 16
_NW = _NC * _NS

_LN_EPS = 1e-5


def _ln(h, g, b):
    mu = jnp.mean(h, axis=-1, keepdims=True)
    var = jnp.mean((h - mu) ** 2, axis=-1, keepdims=True)
    return (h - mu) / jnp.sqrt(var + _LN_EPS) * g + b


def _node_kernel(x_ref, posp_ref, Wf_ref, bf_ref, gf_ref, bef_ref, Ww1_ref,
                 Wq_ref, bq_ref, si_ref, sj_ref):
    x = x_ref[...]
    h = jnp.maximum(
        jnp.dot(x, Wf_ref[...], preferred_element_type=jnp.float32)
        + bf_ref[...], 0.0)
    f = _ln(h, gf_ref[...], bef_ref[...])
    fw = jnp.dot(f, Ww1_ref[...], preferred_element_type=jnp.float32)
    q = jnp.dot(f, Wq_ref[...], preferred_element_type=jnp.float32) + bq_ref[...]
    posp = posp_ref[...]
    si_ref[:, 0:128] = fw
    si_ref[:, 128:256] = q
    si_ref[:, 256:272] = posp
    sj_ref[:, 0:128] = fw
    sj_ref[:, 128:144] = posp


def _edge_kernel(gi_ref, gj_ref, Ww2_ref, bwf_ref, gwf_ref, bewf_ref,
                 Wpp_ref, bp_ref, gp_ref, bep_ref, Wk_ref, bk_ref, p_ref):
    gi = gi_ref[...]
    gj = gj_ref[...]
    g = gi[:, 0:128] - gj[:, 0:128]
    qi = gi[:, 128:256]
    dp = gi[:, 256:272] - gj[:, 128:144]
    h = g + jnp.dot(dp, Ww2_ref[...], preferred_element_type=jnp.float32) \
        + bwf_ref[...]
    wij = _ln(jnp.maximum(h, 0.0), gwf_ref[...], bewf_ref[...])
    pe = jnp.dot(dp, Wpp_ref[...], preferred_element_type=jnp.float32) \
        + bp_ref[...]
    pe = _ln(jnp.maximum(pe, 0.0), gp_ref[...], bep_ref[...])
    key = jnp.dot(wij, Wk_ref[...], preferred_element_type=jnp.float32) \
        + bk_ref[...]
    s = jnp.sum((qi + pe) * key, axis=-1, keepdims=True) * (1.0 / math.sqrt(128.0))
    e = jnp.exp(s)
    p_ref[:, 0:128] = e * wij
    p_ref[:, 128:144] = jnp.broadcast_to(e, (e.shape[0], 16))


def _final_kernel(acc_ref, x_ref, gfin_ref, befin_ref, o_ref):
    a = acc_ref[...]
    numer = a[0, :, 0:128] + a[1, :, 0:128]
    den = a[0, :, 128:129] + a[1, :, 128:129]
    agg = numer / (den + 1e-16)
    o_ref[...] = _ln(agg + x_ref[...], gfin_ref[...], befin_ref[...])


def _make_gather(N, E, ch, nch):
    epw = E // _NW
    mesh = plsc.VectorSubcoreMesh(core_axis_name="c", subcore_axis_name="s",
                                  num_cores=_NC, num_subcores=_NS)

    @functools.partial(
        pl.kernel, mesh=mesh,
        out_type=[jax.ShapeDtypeStruct((E, 272), jnp.float32),
                  jax.ShapeDtypeStruct((E, 144), jnp.float32)],
        scratch_types=[pltpu.VMEM((nch, ch), jnp.int32),
                       pltpu.VMEM((nch, ch), jnp.int32),
                       pltpu.VMEM((ch, 272), jnp.float32),
                       pltpu.VMEM((ch, 144), jnp.float32),
                       pltpu.SemaphoreType.DMA,
                       pltpu.SemaphoreType.DMA],
        compiler_params=pltpu.CompilerParams(use_tc_tiling_on_sc=False))
    def gather(si_hbm, sj_hbm, src_hbm, dst_hbm, gi_hbm, gj_hbm,
               idxs_v, idxd_v, gi_v, gj_v, sem1, sem2):
        wid = lax.axis_index("s") * _NC + lax.axis_index("c")
        pltpu.sync_copy(src_hbm.at[wid], idxs_v)
        pltpu.sync_copy(dst_hbm.at[wid], idxd_v)
        base_w = wid * epw

        def body(c, carry):
            base = base_w + c * ch
            d1 = pltpu.async_copy(si_hbm.at[idxd_v.at[c]], gi_v, sem1)
            d2 = pltpu.async_copy(sj_hbm.at[idxs_v.at[c]], gj_v, sem2)
            d1.wait()
            d2.wait()
            d3 = pltpu.async_copy(gi_v, gi_hbm.at[pl.ds(base, ch)], sem1)
            d4 = pltpu.async_copy(gj_v, gj_hbm.at[pl.ds(base, ch)], sem2)
            d3.wait()
            d4.wait()
            return carry

        lax.fori_loop(0, nch, body, 0)

    return gather


def _make_scatter(N, E, ch, nch):
    epw = E // _NW
    npt = N // _NS
    mesh = plsc.VectorSubcoreMesh(core_axis_name="c", subcore_axis_name="s",
                                  num_cores=_NC, num_subcores=_NS)

    @functools.partial(
        pl.kernel, mesh=mesh,
        out_type=jax.ShapeDtypeStruct((_NC, N, 144), jnp.float32),
        scratch_types=[pltpu.VMEM((nch, ch), jnp.int32),
                       pltpu.VMEM((ch, 144), jnp.float32),
                       pltpu.VMEM_SHARED((N, 144), jnp.float32),
                       pltpu.SemaphoreType.DMA],
        compiler_params=pltpu.CompilerParams(use_tc_tiling_on_sc=False))
    def scatter(p_hbm, dst_hbm, zero_hbm, acc_hbm, idx_v, rows_v, shared, sem):
        cid = lax.axis_index("c")
        sid = lax.axis_index("s")
        wid = sid * _NC + cid
        pltpu.sync_copy(zero_hbm.at[pl.ds(sid * npt, npt)],
                        shared.at[pl.ds(sid * npt, npt)])
        plsc.subcore_barrier()
        pltpu.sync_copy(dst_hbm.at[wid], idx_v)
        base_w = wid * epw

        def body(c, carry):
            base = base_w + c * ch
            pltpu.sync_copy(p_hbm.at[pl.ds(base, ch)], rows_v)
            pltpu.sync_copy(rows_v, shared.at[idx_v.at[c]], add=True)
            return carry

        lax.fori_loop(0, nch, body, 0)
        plsc.subcore_barrier()
        pltpu.sync_copy(shared.at[pl.ds(sid * npt, npt)],
                        acc_hbm.at[cid, pl.ds(sid * npt, npt)])

    return scatter


def kernel(x, pos, edge_index, W_feat, b_feat, g_feat, be_feat, W_wf, b_wf,
           g_wf, be_wf, W_q, b_q, W_k, b_k, W_pos, b_pos, g_pos, be_pos,
           g_fin, be_fin):
    N, D = x.shape
    E = edge_index.shape[1]
    ch = 80
    nch = (E // _NW) // ch

    posp = jnp.pad(pos, ((0, 0), (0, 13)))
    Ww1 = W_wf[:D]
    Ww2 = jnp.pad(W_wf[D:], ((0, 13), (0, 0)))
    Wpp = jnp.pad(W_pos, ((0, 13), (0, 0)))
    row = lambda v: v.reshape(1, -1)
    src3 = edge_index[0].astype(jnp.int32).reshape(_NW, nch, ch)
    dst3 = edge_index[1].astype(jnp.int32).reshape(_NW, nch, ch)

    bn = 1000
    full = lambda shp: pl.BlockSpec(shp, lambda i: (0, 0))
    si, sj = pl.pallas_call(
        _node_kernel,
        grid=(N // bn,),
        in_specs=[
            pl.BlockSpec((bn, 128), lambda i: (i, 0)),
            pl.BlockSpec((bn, 16), lambda i: (i, 0)),
            full((128, 128)), full((1, 128)), full((1, 128)), full((1, 128)),
            full((128, 128)), full((128, 128)), full((1, 128)),
        ],
        out_specs=[pl.BlockSpec((bn, 272), lambda i: (i, 0)),
                   pl.BlockSpec((bn, 144), lambda i: (i, 0))],
        out_shape=[jax.ShapeDtypeStruct((N, 272), jnp.float32),
                   jax.ShapeDtypeStruct((N, 144), jnp.float32)],
    )(x, posp, W_feat, row(b_feat), row(g_feat), row(be_feat), Ww1,
      W_q, row(b_q))

    gi, gj = _make_gather(N, E, ch, nch)(si, sj, src3, dst3)

    be = 512
    p = pl.pallas_call(
        _edge_kernel,
        grid=(E // be,),
        in_specs=[
            pl.BlockSpec((be, 272), lambda i: (i, 0)),
            pl.BlockSpec((be, 144), lambda i: (i, 0)),
            full((16, 128)), full((1, 128)), full((1, 128)), full((1, 128)),
            full((16, 128)), full((1, 128)), full((1, 128)), full((1, 128)),
            full((128, 128)), full((1, 128)),
        ],
        out_specs=pl.BlockSpec((be, 144), lambda i: (i, 0)),
        out_shape=jax.ShapeDtypeStruct((E, 144), jnp.float32),
    )(gi, gj, Ww2, row(b_wf), row(g_wf), row(be_wf), Wpp, row(b_pos),
      row(g_pos), row(be_pos), W_k, row(b_k))

    zero = jnp.zeros((N, 144), jnp.float32)
    acc = _make_scatter(N, E, ch, nch)(p, dst3, zero)

    out = pl.pallas_call(
        _final_kernel,
        grid=(N // bn,),
        in_specs=[
            pl.BlockSpec((_NC, bn, 144), lambda i: (0, i, 0)),
            pl.BlockSpec((bn, 128), lambda i: (i, 0)),
            full((1, 128)), full((1, 128)),
        ],
        out_specs=pl.BlockSpec((bn, 128), lambda i: (i, 0)),
        out_shape=jax.ShapeDtypeStruct((N, 128), jnp.float32),
    )(acc, x, row(g_fin), row(be_fin))
    return out


# speedup vs baseline: 4.7654x; 1.3895x over previous
import functools
import math

import jax
import jax.numpy as jnp
from jax import lax
from jax.experimental import pallas as pl
from jax.experimental.pallas import tpu as pltpu
import jax.experimental.pallas.tpu_sc as plsc

_NC = 2
_NS = 16
_NW = _NC * _NS

_LN_EPS = 1e-5


def _ln(h, g, b):
    mu = jnp.mean(h, axis=-1, keepdims=True)
    var = jnp.mean((h - mu) ** 2, axis=-1, keepdims=True)
    return (h - mu) / jnp.sqrt(var + _LN_EPS) * g + b


def _node_kernel(x_ref, pos_ref, Wf_ref, bf_ref, gf_ref, bef_ref, Ww1_ref,
                 Ww2_ref, Wq_ref, bq_ref, Wp_ref, si_ref, sj_ref):
    x = x_ref[...]
    h = jnp.maximum(
        jnp.dot(x, Wf_ref[...], preferred_element_type=jnp.float32)
        + bf_ref[...], 0.0)
    f = _ln(h, gf_ref[...], bef_ref[...])
    pos = pos_ref[...]
    fw = jnp.dot(f, Ww1_ref[...], preferred_element_type=jnp.float32) \
        + jnp.dot(pos, Ww2_ref[...], preferred_element_type=jnp.float32)
    q = jnp.dot(f, Wq_ref[...], preferred_element_type=jnp.float32) + bq_ref[...]
    pp = jnp.dot(pos, Wp_ref[...], preferred_element_type=jnp.float32)
    si_ref[:, 0:128] = fw
    si_ref[:, 128:256] = q
    si_ref[:, 256:384] = pp
    sj_ref[:, 0:128] = fw
    sj_ref[:, 128:256] = pp


def _edge_kernel(gi_ref, gj_ref, bwf_ref, gwf_ref, bewf_ref,
                 bp_ref, gp_ref, bep_ref, Wk_ref, bk_ref, p_ref, e_ref):
    gi = gi_ref[...]
    gj = gj_ref[...]
    h = gi[:, 0:128] - gj[:, 0:128] + bwf_ref[...]
    wij = _ln(jnp.maximum(h, 0.0), gwf_ref[...], bewf_ref[...])
    pe = gi[:, 256:384] - gj[:, 128:256] + bp_ref[...]
    pe = _ln(jnp.maximum(pe, 0.0), gp_ref[...], bep_ref[...])
    key = jnp.dot(wij, Wk_ref[...], preferred_element_type=jnp.float32) \
        + bk_ref[...]
    qi = gi[:, 128:256]
    s = jnp.sum((qi + pe) * key, axis=-1, keepdims=True) * (1.0 / math.sqrt(128.0))
    e = jnp.exp(s)
    p_ref[...] = e * wij
    e_ref[...] = e


def _final_kernel(acc_ref, dn_ref, x_ref, gfin_ref, befin_ref, o_ref):
    a = acc_ref[...]
    numer = a[0] + a[1]
    den = jnp.sum(dn_ref[...], axis=-1, keepdims=True)
    agg = numer / (den + 1e-16)
    o_ref[...] = _ln(agg + x_ref[...], gfin_ref[...], befin_ref[...])


def _make_gather(N, E, ch, nch):
    epw = E // _NW
    mesh = plsc.VectorSubcoreMesh(core_axis_name="c", subcore_axis_name="s",
                                  num_cores=_NC, num_subcores=_NS)

    @functools.partial(
        pl.kernel, mesh=mesh,
        out_type=[jax.ShapeDtypeStruct((E, 384), jnp.float32),
                  jax.ShapeDtypeStruct((E, 256), jnp.float32)],
        scratch_types=[pltpu.VMEM((ch,), jnp.int32),
                       pltpu.VMEM((ch,), jnp.int32),
                       pltpu.VMEM((ch, 384), jnp.float32),
                       pltpu.VMEM((ch, 256), jnp.float32),
                       pltpu.SemaphoreType.DMA,
                       pltpu.SemaphoreType.DMA],
        compiler_params=pltpu.CompilerParams(needs_layout_passes=False))
    def gather(si_hbm, sj_hbm, src_hbm, dst_hbm, gi_hbm, gj_hbm,
               idxs_v, idxd_v, gi_v, gj_v, sem1, sem2):
        wid = lax.axis_index("s") * _NC + lax.axis_index("c")
        base_w = wid * epw

        def body(c, carry):
            base = base_w + c * ch
            pltpu.sync_copy(src_hbm.at[wid, c], idxs_v)
            pltpu.sync_copy(dst_hbm.at[wid, c], idxd_v)
            d1 = pltpu.async_copy(si_hbm.at[idxd_v], gi_v, sem1)
            d2 = pltpu.async_copy(sj_hbm.at[idxs_v], gj_v, sem2)
            d1.wait()
            d2.wait()
            d3 = pltpu.async_copy(gi_v, gi_hbm.at[pl.ds(base, ch)], sem1)
            d4 = pltpu.async_copy(gj_v, gj_hbm.at[pl.ds(base, ch)], sem2)
            d3.wait()
            d4.wait()
            return carry

        lax.fori_loop(0, nch, body, 0)

    return gather


def _make_scatter(N, E, ch, nch):
    epw = E // _NW
    npt = 624
    rem = N - npt * _NS
    mesh = plsc.VectorSubcoreMesh(core_axis_name="c", subcore_axis_name="s",
                                  num_cores=_NC, num_subcores=_NS)

    @functools.partial(
        pl.kernel, mesh=mesh,
        out_type=[jax.ShapeDtypeStruct((_NC, N, 128), jnp.float32),
                  jax.ShapeDtypeStruct((_NW, N), jnp.float32)],
        scratch_types=[pltpu.VMEM((ch,), jnp.int32),
                       pltpu.VMEM((ch,), jnp.float32),
                       pltpu.VMEM((ch, 128), jnp.float32),
                       pltpu.VMEM((N,), jnp.float32),
                       pltpu.VMEM_SHARED((N, 128), jnp.float32),
                       pltpu.SemaphoreType.DMA],
        compiler_params=pltpu.CompilerParams(needs_layout_passes=False))
    def scatter(p_hbm, e_hbm, dst_hbm, zero_hbm, acc_hbm, dnt_hbm,
                ic_v, ev_v, rows_v, dn_v, shared, sem):
        cid = lax.axis_index("c")
        sid = lax.axis_index("s")
        wid = sid * _NC + cid
        pltpu.sync_copy(zero_hbm.at[pl.ds(sid * npt, npt)],
                        shared.at[pl.ds(sid * npt, npt)])

        @pl.when(sid == 0)
        def _():
            pltpu.sync_copy(zero_hbm.at[pl.ds(_NS * npt, rem)],
                            shared.at[pl.ds(_NS * npt, rem)])

        def zbody(k, carry):
            dn_v[pl.ds(k * 16, 16)] = jnp.zeros((16,), jnp.float32)
            return carry

        lax.fori_loop(0, N // 16, zbody, 0)
        plsc.subcore_barrier()
        base_w = wid * epw

        def body(c, carry):
            base = base_w + c * ch
            pltpu.sync_copy(dst_hbm.at[wid, c], ic_v)
            pltpu.sync_copy(e_hbm.at[wid, c], ev_v)
            pltpu.sync_copy(p_hbm.at[pl.ds(base, ch)], rows_v)
            pltpu.sync_copy(rows_v, shared.at[ic_v], add=True)

            def dbody(k, carry2):
                iv = ic_v[pl.ds(k * 16, 16)]
                ev = ev_v[pl.ds(k * 16, 16)]
                plsc.addupdate_scatter(dn_v, [iv], ev)
                return carry2

            lax.fori_loop(0, ch // 16, dbody, 0)
            return carry

        lax.fori_loop(0, nch, body, 0)
        plsc.subcore_barrier()
        pltpu.sync_copy(shared.at[pl.ds(sid * npt, npt)],
                        acc_hbm.at[cid, pl.ds(sid * npt, npt)])

        @pl.when(sid == 0)
        def _():
            pltpu.sync_copy(shared.at[pl.ds(_NS * npt, rem)],
                            acc_hbm.at[cid, pl.ds(_NS * npt, rem)])

        pltpu.sync_copy(dn_v, dnt_hbm.at[wid])

    return scatter


def kernel(x, pos, edge_index, W_feat, b_feat, g_feat, be_feat, W_wf, b_wf,
           g_wf, be_wf, W_q, b_q, W_k, b_k, W_pos, b_pos, g_pos, be_pos,
           g_fin, be_fin):
    N, D = x.shape
    E = edge_index.shape[1]
    ch = 80
    nch = (E // _NW) // ch

    Ww1 = W_wf[:D]
    Ww2 = W_wf[D:]
    row = lambda v: v.reshape(1, -1)
    src3 = edge_index[0].astype(jnp.int32).reshape(_NW, nch, ch)
    dst3 = edge_index[1].astype(jnp.int32).reshape(_NW, nch, ch)

    bn = 1000
    full = lambda shp: pl.BlockSpec(shp, lambda i: (0, 0))
    si, sj = pl.pallas_call(
        _node_kernel,
        grid=(N // bn,),
        in_specs=[
            pl.BlockSpec((bn, 128), lambda i: (i, 0)),
            pl.BlockSpec((bn, 3), lambda i: (i, 0)),
            full((128, 128)), full((1, 128)), full((1, 128)), full((1, 128)),
            full((128, 128)), full((3, 128)), full((128, 128)), full((1, 128)),
            full((3, 128)),
        ],
        out_specs=[pl.BlockSpec((bn, 384), lambda i: (i, 0)),
                   pl.BlockSpec((bn, 256), lambda i: (i, 0))],
        out_shape=[jax.ShapeDtypeStruct((N, 384), jnp.float32),
                   jax.ShapeDtypeStruct((N, 256), jnp.float32)],
    )(x, pos, W_feat, row(b_feat), row(g_feat), row(be_feat), Ww1, Ww2,
      W_q, row(b_q), W_pos)

    gi, gj = _make_gather(N, E, ch, nch)(si, sj, src3, dst3)

    be = 512
    p, ev = pl.pallas_call(
        _edge_kernel,
        grid=(E // be,),
        in_specs=[
            pl.BlockSpec((be, 384), lambda i: (i, 0)),
            pl.BlockSpec((be, 256), lambda i: (i, 0)),
            full((1, 128)), full((1, 128)), full((1, 128)),
            full((1, 128)), full((1, 128)), full((1, 128)),
            full((128, 128)), full((1, 128)),
        ],
        out_specs=[pl.BlockSpec((be, 128), lambda i: (i, 0)),
                   pl.BlockSpec((be, 1), lambda i: (i, 0))],
        out_shape=[jax.ShapeDtypeStruct((E, 128), jnp.float32),
                   jax.ShapeDtypeStruct((E, 1), jnp.float32)],
    )(gi, gj, row(b_wf), row(g_wf), row(be_wf), row(b_pos),
      row(g_pos), row(be_pos), W_k, row(b_k))

    zero = jnp.zeros((N, 128), jnp.float32)
    e3 = ev.reshape(_NW, nch, ch)
    acc, dnt = _make_scatter(N, E, ch, nch)(p, e3, dst3, zero)
    dnt = dnt.T

    out = pl.pallas_call(
        _final_kernel,
        grid=(N // bn,),
        in_specs=[
            pl.BlockSpec((_NC, bn, 128), lambda i: (0, i, 0)),
            pl.BlockSpec((bn, _NW), lambda i: (i, 0)),
            pl.BlockSpec((bn, 128), lambda i: (i, 0)),
            full((1, 128)), full((1, 128)),
        ],
        out_specs=pl.BlockSpec((bn, 128), lambda i: (i, 0)),
        out_shape=jax.ShapeDtypeStruct((N, 128), jnp.float32),
    )(acc, dnt, x, row(g_fin), row(be_fin))
    return out
